# Initial kernel scaffold; baseline (speedup 1.0000x reference)
import jax
import jax.numpy as jnp
from jax.experimental import pallas as pl

S = 7
B = 2
C = 20
HW = S * S
MAXOBJ = 16
IGNORE_THRESH = 0.5
L_COORD = 5.0
L_OBJ = 1.0
L_NOOBJ = 0.5
L_CLASS = 1.0
NB = 128


def _sig(x):
    return 1.0 / (1.0 + jnp.exp(-x))


def _loss_kernel(op_ref, tg_ref, out_ref):
    f32 = jnp.float32
    nb = op_ref.shape[-1]
    cell_ids = jax.lax.broadcasted_iota(f32, (HW, 1), 0)
    xs = cell_ids % S
    ys = jnp.floor(cell_ids / S)

    sx, sy, sw, sh = [], [], [], []
    pxm, pxp, pym, pyp, areap, conf = [], [], [], [], [], []
    for b in range(B):
        base = 4 * b
        sxb = _sig(op_ref[base + 0])
        syb = _sig(op_ref[base + 1])
        swb = _sig(op_ref[base + 2])
        shb = _sig(op_ref[base + 3])
        sx.append(sxb)
        sy.append(syb)
        sw.append(swb)
        sh.append(shb)
        pxb = sxb + xs
        pyb = syb + ys
        pwb = swb * S
        phb = shb * S
        pxm.append(pxb - pwb * 0.5)
        pxp.append(pxb + pwb * 0.5)
        pym.append(pyb - phb * 0.5)
        pyp.append(pyb + phb * 0.5)
        areap.append(pwb * phb)
        conf.append(_sig(op_ref[4 * B + b]))

    tsum = jnp.sum(tg_ref[...], axis=1)
    num_obj = jnp.sum((tsum > 0).astype(f32), axis=0, keepdims=True)
    has_obj = num_obj > 0

    neg_inf = jnp.float32(-jnp.inf)
    miou = [jnp.full((HW, nb), neg_inf, f32) for _ in range(B)]
    boxm = [jnp.zeros((HW, nb), jnp.bool_) for _ in range(B)]
    btx = [jnp.zeros((HW, nb), f32) for _ in range(B)]
    bty = [jnp.zeros((HW, nb), f32) for _ in range(B)]
    btw = [jnp.zeros((HW, nb), f32) for _ in range(B)]
    bth = [jnp.zeros((HW, nb), f32) for _ in range(B)]
    clsm = jnp.zeros((HW, nb), jnp.bool_)
    clst = jnp.zeros((HW, nb), f32)

    for o in range(MAXOBJ):
        g = tg_ref[o]
        gxo = g[0:1] * S
        gyo = g[1:2] * S
        gwo = g[2:3] * S
        gho = g[3:4] * S
        gco = jnp.floor(g[4:5])
        v = num_obj > o
        gx1 = gxo - gwo * 0.5
        gx2 = gxo + gwo * 0.5
        gy1 = gyo - gho * 0.5
        gy2 = gyo + gho * 0.5
        areag = gwo * gho
        ious = []
        for b in range(B):
            tlx = jnp.maximum(pxm[b], gx1)
            brx = jnp.minimum(pxp[b], gx2)
            tly = jnp.maximum(pym[b], gy1)
            bry = jnp.minimum(pyp[b], gy2)
            en = jnp.logical_and(tlx < brx, tly < bry)
            ai = jnp.where(en, (brx - tlx) * (bry - tly), 0.0)
            iou_bo = ai / (areap[b] + areag - ai)
            ious.append(iou_bo)
            miou[b] = jnp.maximum(miou[b], jnp.where(v, iou_bo, neg_inf))
        cxo = jnp.floor(gxo)
        cyo = jnp.floor(gyo)
        cello = cyo * S + cxo
        hit = cell_ids == cello
        hv = jnp.logical_and(hit, v)
        slot1 = ious[1] > ious[0]
        clsm = jnp.logical_or(clsm, hv)
        clst = jnp.where(hv, gco, clst)
        txo = gxo - cxo
        tyo = gyo - cyo
        two = gwo / S
        tho = gho / S
        for b in range(B):
            if b == 0:
                selb = jnp.logical_and(hv, jnp.logical_not(slot1))
            else:
                selb = jnp.logical_and(hv, slot1)
            boxm[b] = jnp.logical_or(boxm[b], selb)
            btx[b] = jnp.where(selb, txo, btx[b])
            bty[b] = jnp.where(selb, tyo, bty[b])
            btw[b] = jnp.where(selb, two, btw[b])
            bth[b] = jnp.where(selb, tho, bth[b])

    npos = jnp.zeros((1, nb), f32)
    for b in range(B):
        npos += jnp.sum((miou[b] > IGNORE_THRESH).astype(f32), axis=0, keepdims=True)
    anypos = npos > 0

    lobj = jnp.float32(0.0)
    lnoobj = jnp.float32(0.0)
    lxy = jnp.float32(0.0)
    lwh = jnp.float32(0.0)
    for b in range(B):
        keep_b = jnp.logical_and(
            has_obj,
            jnp.logical_not(jnp.logical_and(anypos, miou[b] >= IGNORE_THRESH)),
        )
        noobj_b = jnp.logical_and(keep_b, jnp.logical_not(boxm[b]))
        lobj += jnp.sum(jnp.where(boxm[b], (conf[b] - miou[b]) ** 2, 0.0))
        lnoobj += jnp.sum(jnp.where(noobj_b, conf[b] * conf[b], 0.0))
        lxy += jnp.sum(
            jnp.where(boxm[b], (sx[b] - btx[b]) ** 2 + (sy[b] - bty[b]) ** 2, 0.0)
        )
        lwh += jnp.sum(
            jnp.where(
                boxm[b],
                (jnp.sqrt(sw[b]) - jnp.sqrt(btw[b])) ** 2
                + (jnp.sqrt(sh[b]) - jnp.sqrt(bth[b])) ** 2,
                0.0,
            )
        )

    m = op_ref[5 * B]
    for ch in range(1, C):
        m = jnp.maximum(m, op_ref[5 * B + ch])
    ssum = jnp.zeros((HW, nb), f32)
    psel = jnp.zeros((HW, nb), f32)
    for ch in range(C):
        p = op_ref[5 * B + ch]
        ssum += jnp.exp(p - m)
        psel = jnp.where(clst == ch, p, psel)
    picked = psel - m - jnp.log(ssum)
    lclass = -jnp.sum(jnp.where(clsm, picked, 0.0))

    total = (lxy + lwh) * L_COORD + lobj * L_OBJ + lnoobj * L_NOOBJ + lclass * L_CLASS

    @pl.when(pl.program_id(0) == 0)
    def _():
        out_ref[0, 0] = 0.0

    out_ref[0, 0] += total


def _run(opt, tgt, interpret=False):
    n = opt.shape[-1]
    nb = min(NB, n)
    grid = n // nb
    out = pl.pallas_call(
        _loss_kernel,
        grid=(grid,),
        in_specs=[
            pl.BlockSpec((5 * B + C, HW, nb), lambda i: (0, 0, i)),
            pl.BlockSpec((MAXOBJ, 5, nb), lambda i: (0, 0, i)),
        ],
        out_specs=pl.BlockSpec((1, 1), lambda i: (0, 0)),
        out_shape=jax.ShapeDtypeStruct((1, 1), jnp.float32),
        interpret=interpret,
    )(opt, tgt)
    return out[0, 0]


@jax.jit
def kernel(outputs, targets):
    n = outputs.shape[0]
    opt = jnp.transpose(outputs.reshape(n, 5 * B + C, HW), (1, 2, 0))
    tgt = jnp.transpose(targets, (1, 2, 0))
    return _run(opt, tgt)


# speedup vs baseline: 269.4625x; 269.4625x over previous
import jax
import jax.numpy as jnp
from jax.experimental import pallas as pl
from jax.experimental.pallas import tpu as pltpu

S = 7
B = 2
C = 20
HW = S * S
MAXOBJ = 16
IGNORE_THRESH = 0.5
L_COORD = 5.0
L_OBJ = 1.0
L_NOOBJ = 0.5
L_CLASS = 1.0
NB = 128


def _sig(x):
    return 1.0 / (1.0 + jnp.exp(-x))


def _loss_kernel(op_ref, tg_ref, out_ref):
    f32 = jnp.float32
    nb = op_ref.shape[-1]
    cell_ids = jax.lax.broadcasted_iota(jnp.int32, (HW, 1), 0).astype(f32)
    xs = cell_ids % S
    ys = jnp.floor(cell_ids / S)

    sx, sy, sw, sh = [], [], [], []
    pxm, pxp, pym, pyp, areap, conf = [], [], [], [], [], []
    for b in range(B):
        base = 4 * b
        sxb = _sig(op_ref[base + 0])
        syb = _sig(op_ref[base + 1])
        swb = _sig(op_ref[base + 2])
        shb = _sig(op_ref[base + 3])
        sx.append(sxb)
        sy.append(syb)
        sw.append(swb)
        sh.append(shb)
        pxb = sxb + xs
        pyb = syb + ys
        pwb = swb * S
        phb = shb * S
        pxm.append(pxb - pwb * 0.5)
        pxp.append(pxb + pwb * 0.5)
        pym.append(pyb - phb * 0.5)
        pyp.append(pyb + phb * 0.5)
        areap.append(pwb * phb)
        conf.append(_sig(op_ref[4 * B + b]))

    tsum = jnp.sum(tg_ref[...], axis=1)
    num_obj = jnp.sum((tsum > 0).astype(f32), axis=0, keepdims=True)
    has_obj = num_obj > 0

    neg_inf = jnp.float32(-jnp.inf)
    miou = [jnp.full((HW, nb), neg_inf, f32) for _ in range(B)]
    boxm = [jnp.zeros((HW, nb), jnp.bool_) for _ in range(B)]
    btx = [jnp.zeros((HW, nb), f32) for _ in range(B)]
    bty = [jnp.zeros((HW, nb), f32) for _ in range(B)]
    btw = [jnp.zeros((HW, nb), f32) for _ in range(B)]
    bth = [jnp.zeros((HW, nb), f32) for _ in range(B)]
    clsm = jnp.zeros((HW, nb), jnp.bool_)
    clst = jnp.zeros((HW, nb), f32)

    for o in range(MAXOBJ):
        g = tg_ref[o]
        gxo = g[0:1] * S
        gyo = g[1:2] * S
        gwo = g[2:3] * S
        gho = g[3:4] * S
        gco = jnp.floor(g[4:5])
        v = num_obj > o
        gx1 = gxo - gwo * 0.5
        gx2 = gxo + gwo * 0.5
        gy1 = gyo - gho * 0.5
        gy2 = gyo + gho * 0.5
        areag = gwo * gho
        ious = []
        for b in range(B):
            tlx = jnp.maximum(pxm[b], gx1)
            brx = jnp.minimum(pxp[b], gx2)
            tly = jnp.maximum(pym[b], gy1)
            bry = jnp.minimum(pyp[b], gy2)
            en = jnp.logical_and(tlx < brx, tly < bry)
            ai = jnp.where(en, (brx - tlx) * (bry - tly), 0.0)
            iou_bo = ai / (areap[b] + areag - ai)
            ious.append(iou_bo)
            miou[b] = jnp.maximum(miou[b], jnp.where(v, iou_bo, neg_inf))
        cxo = jnp.floor(gxo)
        cyo = jnp.floor(gyo)
        cello = cyo * S + cxo
        hit = cell_ids == cello
        hv = jnp.logical_and(hit, v)
        slot1 = ious[1] > ious[0]
        clsm = jnp.logical_or(clsm, hv)
        clst = jnp.where(hv, gco, clst)
        txo = gxo - cxo
        tyo = gyo - cyo
        two = gwo / S
        tho = gho / S
        for b in range(B):
            if b == 0:
                selb = jnp.logical_and(hv, jnp.logical_not(slot1))
            else:
                selb = jnp.logical_and(hv, slot1)
            boxm[b] = jnp.logical_or(boxm[b], selb)
            btx[b] = jnp.where(selb, txo, btx[b])
            bty[b] = jnp.where(selb, tyo, bty[b])
            btw[b] = jnp.where(selb, two, btw[b])
            bth[b] = jnp.where(selb, tho, bth[b])

    npos = jnp.zeros((1, nb), f32)
    for b in range(B):
        npos += jnp.sum((miou[b] > IGNORE_THRESH).astype(f32), axis=0, keepdims=True)
    anypos = npos > 0

    lobj = jnp.float32(0.0)
    lnoobj = jnp.float32(0.0)
    lxy = jnp.float32(0.0)
    lwh = jnp.float32(0.0)
    for b in range(B):
        keep_b = jnp.logical_and(
            has_obj,
            jnp.logical_not(jnp.logical_and(anypos, miou[b] >= IGNORE_THRESH)),
        )
        noobj_b = jnp.logical_and(keep_b, jnp.logical_not(boxm[b]))
        lobj += jnp.sum(jnp.where(boxm[b], (conf[b] - miou[b]) ** 2, 0.0))
        lnoobj += jnp.sum(jnp.where(noobj_b, conf[b] * conf[b], 0.0))
        lxy += jnp.sum(
            jnp.where(boxm[b], (sx[b] - btx[b]) ** 2 + (sy[b] - bty[b]) ** 2, 0.0)
        )
        lwh += jnp.sum(
            jnp.where(
                boxm[b],
                (jnp.sqrt(sw[b]) - jnp.sqrt(btw[b])) ** 2
                + (jnp.sqrt(sh[b]) - jnp.sqrt(bth[b])) ** 2,
                0.0,
            )
        )

    m = op_ref[5 * B]
    for ch in range(1, C):
        m = jnp.maximum(m, op_ref[5 * B + ch])
    ssum = jnp.zeros((HW, nb), f32)
    psel = jnp.zeros((HW, nb), f32)
    for ch in range(C):
        p = op_ref[5 * B + ch]
        ssum += jnp.exp(p - m)
        psel = jnp.where(clst == ch, p, psel)
    picked = psel - m - jnp.log(ssum)
    lclass = -jnp.sum(jnp.where(clsm, picked, 0.0))

    total = (lxy + lwh) * L_COORD + lobj * L_OBJ + lnoobj * L_NOOBJ + lclass * L_CLASS

    @pl.when(pl.program_id(0) == 0)
    def _():
        out_ref[0, 0] = 0.0

    out_ref[0, 0] += total


def _run(opt, tgt, interpret=False):
    n = opt.shape[-1]
    nb = min(NB, n)
    grid = n // nb
    out = pl.pallas_call(
        _loss_kernel,
        grid=(grid,),
        in_specs=[
            pl.BlockSpec((5 * B + C, HW, nb), lambda i: (0, 0, i)),
            pl.BlockSpec((MAXOBJ, 5, nb), lambda i: (0, 0, i)),
        ],
        out_specs=pl.BlockSpec(memory_space=pltpu.SMEM),
        out_shape=jax.ShapeDtypeStruct((1, 1), jnp.float32),
        interpret=interpret,
    )(opt, tgt)
    return out[0, 0]


@jax.jit
def kernel(outputs, targets):
    n = outputs.shape[0]
    opt = jnp.transpose(outputs.reshape(n, 5 * B + C, HW), (1, 2, 0))
    tgt = jnp.transpose(targets, (1, 2, 0))
    return _run(opt, tgt)
